# baseline (speedup 1.0000x reference)
import functools

import jax
import jax.numpy as jnp
from jax import lax
from jax.experimental import pallas as pl
from jax.experimental.pallas import tpu as pltpu
from jax.experimental.pallas import tpu_sc as plsc

T = 2048
D = 2048
H = 1024
E = 8
BT = 128
NB = T // BT + E
NPAD = NB * BT

NC = 2
NS = 16
NW = NC * NS
TPW = T // NW
CH = 16
SW = 128
NCH = TPW // CH


def _dotT(a, b, out_dtype=jnp.float32):
    return lax.dot_general(a, b, (((1,), (1,)), ((), ())),
                           preferred_element_type=out_dtype)


def _router_body(xf_ref, gate_ref, bias_ref, pos_ref, sc16_ref, cnt_ref,
                 blk_e_ref, blk_a_ref, oh_ref, rank_ref):
    xf = xf_ref[...]
    logits = _dotT(xf, gate_ref[...])
    scores = jax.nn.sigmoid(logits)
    biased = scores + bias_ref[...]
    lane = lax.broadcasted_iota(jnp.int32, (T, E), 1)
    mx = jnp.max(biased, axis=1, keepdims=True)
    sel = jnp.min(jnp.where(biased >= mx, lane, E), axis=1, keepdims=True)
    oh = (lane == sel).astype(jnp.float32)
    oh_ref[...] = oh
    score_sel = jnp.sum(oh * scores, axis=1, keepdims=True)
    sc16_ref[...] = score_sel * jnp.ones((1, SW), jnp.float32)

    r = lax.broadcasted_iota(jnp.int32, (BT, BT), 0)
    c = lax.broadcasted_iota(jnp.int32, (BT, BT), 1)
    ltri = (c < r).astype(jnp.float32)

    def chunk(i, carry):
        ohc = oh_ref[pl.ds(i * BT, BT), :]
        rankc = lax.dot_general(ltri, ohc, (((1,), (0,)), ((), ())),
                                preferred_element_type=jnp.float32)
        rankc = rankc + carry
        rank_ref[pl.ds(i * BT, BT), :] = jnp.sum(
            rankc * ohc, axis=1, keepdims=True)
        return carry + jnp.sum(ohc, axis=0, keepdims=True)

    counts_f = lax.fori_loop(0, T // BT, chunk, jnp.zeros((1, E), jnp.float32))
    cnt_ref[...] = counts_f.astype(jnp.int32)

    nblk = jnp.floor((counts_f + (BT - 1)) * (1.0 / BT))
    r8 = lax.broadcasted_iota(jnp.int32, (E, E), 0)
    c8 = lax.broadcasted_iota(jnp.int32, (E, E), 1)
    sut = (r8 < c8).astype(jnp.float32)
    offs = lax.dot_general(nblk, sut, (((1,), (0,)), ((), ())),
                           preferred_element_type=jnp.float32)
    total = jnp.sum(nblk, axis=1, keepdims=True)

    pos_f = jnp.sum(oh_ref[...] * (offs * float(BT)), axis=1,
                    keepdims=True) + rank_ref[...]
    pos_ref[...] = pos_f.astype(jnp.int32)

    jblk = lax.broadcasted_iota(jnp.int32, (NB, E), 0).astype(jnp.float32)
    blk_e_ref[...] = (jnp.sum((jblk >= offs).astype(jnp.float32), axis=1,
                              keepdims=True) - 1.0).astype(jnp.int32)
    jcol = lax.broadcasted_iota(jnp.int32, (NB, 1), 0).astype(jnp.float32)
    blk_a_ref[...] = (jcol < total).astype(jnp.int32)


def _router(xf, gate, bias2d):
    return pl.pallas_call(
        _router_body,
        out_shape=[
            jax.ShapeDtypeStruct((T, 1), jnp.int32),
            jax.ShapeDtypeStruct((T, SW), jnp.float32),
            jax.ShapeDtypeStruct((1, E), jnp.int32),
            jax.ShapeDtypeStruct((NB, 1), jnp.int32),
            jax.ShapeDtypeStruct((NB, 1), jnp.int32),
        ],
        scratch_shapes=[
            pltpu.VMEM((T, E), jnp.float32),
            pltpu.VMEM((T, 1), jnp.float32),
        ],
    )(xf, gate, bias2d)


def _dispatch_body(pos_hbm, xf_hbm, sc_hbm, xs_out, sc_out,
                   idx_v, rows_v, srows_v, sem):
    wid = lax.axis_index("s") * NC + lax.axis_index("c")
    pltpu.sync_copy(pos_hbm.at[wid], idx_v)
    for ci in range(NCH):
        base = wid * TPW + ci * CH
        pltpu.sync_copy(xf_hbm.at[pl.ds(base, CH)], rows_v)
        pltpu.async_copy(rows_v, xs_out.at[idx_v.at[ci]], sem).wait()
        pltpu.sync_copy(sc_hbm.at[pl.ds(base, CH)], srows_v)
        pltpu.async_copy(srows_v, sc_out.at[idx_v.at[ci]], sem).wait()


def _dispatch(pos3, xf, sc16):
    return pl.kernel(
        _dispatch_body,
        out_type=[
            jax.ShapeDtypeStruct((NPAD, D), jnp.float32),
            jax.ShapeDtypeStruct((NPAD, SW), jnp.float32),
        ],
        mesh=plsc.VectorSubcoreMesh(core_axis_name="c", subcore_axis_name="s"),
        scratch_types=[
            pltpu.VMEM((NCH, CH), jnp.int32),
            pltpu.VMEM((CH, D), jnp.float32),
            pltpu.VMEM((CH, SW), jnp.float32),
            pltpu.SemaphoreType.DMA,
        ],
    )(pos3, xf, sc16)


def _ffn_body(be_ref, act_ref, xs_ref, sc_ref, w1_ref, w3_ref, w2_ref, or_ref):
    j = pl.program_id(0)

    @pl.when(act_ref[j] != 0)
    def _():
        x = xs_ref[...] * sc_ref[:, 0:1]
        rb = x.astype(jnp.bfloat16)
        h1 = _dotT(rb, w1_ref[0]).astype(jnp.bfloat16)
        g3 = _dotT(rb, w3_ref[0]).astype(jnp.bfloat16)
        h = h1 * jax.nn.sigmoid(h1) * g3
        o = _dotT(h, w2_ref[0]).astype(jnp.bfloat16)
        or_ref[...] = o.astype(jnp.float32)


def _ffn(blk_e, blk_a, xs_pad, sc_pad, w1b, w3b, w2b):
    grid_spec = pltpu.PrefetchScalarGridSpec(
        num_scalar_prefetch=2,
        grid=(NB,),
        in_specs=[
            pl.BlockSpec((BT, D), lambda j, be, act: (j, 0)),
            pl.BlockSpec((BT, SW), lambda j, be, act: (j, 0)),
            pl.BlockSpec((1, H, D), lambda j, be, act: (be[j], 0, 0)),
            pl.BlockSpec((1, H, D), lambda j, be, act: (be[j], 0, 0)),
            pl.BlockSpec((1, D, H), lambda j, be, act: (be[j], 0, 0)),
        ],
        out_specs=pl.BlockSpec((BT, D), lambda j, be, act: (j, 0)),
    )
    return pl.pallas_call(
        _ffn_body,
        grid_spec=grid_spec,
        out_shape=jax.ShapeDtypeStruct((NPAD, D), jnp.float32),
    )(blk_e, blk_a, xs_pad, sc_pad, w1b, w3b, w2b)


def _unperm_body(pos_hbm, or_hbm, out_hbm, idx_v, rows_v, sem):
    wid = lax.axis_index("s") * NC + lax.axis_index("c")
    pltpu.sync_copy(pos_hbm.at[wid], idx_v)
    for ci in range(NCH):
        base = wid * TPW + ci * CH
        pltpu.async_copy(or_hbm.at[idx_v.at[ci]], rows_v, sem).wait()
        pltpu.sync_copy(rows_v, out_hbm.at[pl.ds(base, CH)])


def _unpermute(pos3, or_pad):
    return pl.kernel(
        _unperm_body,
        out_type=jax.ShapeDtypeStruct((T, D), jnp.float32),
        mesh=plsc.VectorSubcoreMesh(core_axis_name="c", subcore_axis_name="s"),
        scratch_types=[
            pltpu.VMEM((NCH, CH), jnp.int32),
            pltpu.VMEM((CH, D), jnp.float32),
            pltpu.SemaphoreType.DMA,
        ],
    )(pos3, or_pad)


def _shared_body(xf_ref, w1_ref, w3_ref, w2_ref, or_ref, out_ref):
    x = xf_ref[...]
    s1 = _dotT(x, w1_ref[...])
    hs = s1 * jax.nn.sigmoid(s1) * _dotT(x, w3_ref[...])
    out_ref[...] = _dotT(hs.astype(jnp.bfloat16), w2_ref[...]) + or_ref[...]


def _shared(xb, sw1, sw3, sw2, or_tok):
    nblk = T // BT
    return pl.pallas_call(
        _shared_body,
        grid=(nblk,),
        in_specs=[
            pl.BlockSpec((BT, D), lambda j: (j, 0)),
            pl.BlockSpec((H, D), lambda j: (0, 0)),
            pl.BlockSpec((H, D), lambda j: (0, 0)),
            pl.BlockSpec((D, H), lambda j: (0, 0)),
            pl.BlockSpec((BT, D), lambda j: (j, 0)),
        ],
        out_specs=pl.BlockSpec((BT, D), lambda j: (j, 0)),
        out_shape=jax.ShapeDtypeStruct((T, D), jnp.float32),
    )(xb, sw1, sw3, sw2, or_tok)


def kernel(x, gate, expert_bias, w1, w2, w3, shared_w1, shared_w2, shared_w3):
    bs, slen, dim = x.shape
    xf = x.reshape(T, D)

    pos, sc16, counts, blk_e, blk_a = _router(xf, gate,
                                              expert_bias.reshape(1, E))
    pos3 = pos.reshape(NW, NCH, CH)

    xs_pad, sc_pad = _dispatch(pos3, xf, sc16)

    w1b = w1.astype(jnp.bfloat16)
    w3b = w3.astype(jnp.bfloat16)
    w2b = w2.astype(jnp.bfloat16)
    or_pad = _ffn(blk_e.reshape(NB), blk_a.reshape(NB),
                  xs_pad, sc_pad, w1b, w3b, w2b)

    or_tok = _unpermute(pos3, or_pad)

    out = _shared(xf.astype(jnp.bfloat16), shared_w1.astype(jnp.bfloat16),
                  shared_w3.astype(jnp.bfloat16), shared_w2.astype(jnp.bfloat16),
                  or_tok)
    return (out.reshape(bs, slen, dim), counts.reshape(E))


# speedup vs baseline: 1.6309x; 1.6309x over previous
import functools

import jax
import jax.numpy as jnp
from jax import lax
from jax.experimental import pallas as pl
from jax.experimental.pallas import tpu as pltpu
from jax.experimental.pallas import tpu_sc as plsc

T = 2048
D = 2048
H = 1024
E = 8
BT = 256
NB = T // BT + E
NPAD = NB * BT

NC = 2
NS = 16
NW = NC * NS
TPW = T // NW
CH = 16
SW = 128
NCH = TPW // CH


def _dotT(a, b, out_dtype=jnp.float32):
    return lax.dot_general(a, b, (((1,), (1,)), ((), ())),
                           preferred_element_type=out_dtype)


def _router_body(xf_ref, gate_ref, bias_ref, pos_ref, sc16_ref, cnt_ref,
                 blk_e_ref, blk_a_ref, oh_ref, rank_ref):
    xf = xf_ref[...]
    logits = _dotT(xf, gate_ref[...])
    scores = jax.nn.sigmoid(logits)
    biased = scores + bias_ref[...]
    lane = lax.broadcasted_iota(jnp.int32, (T, E), 1)
    mx = jnp.max(biased, axis=1, keepdims=True)
    sel = jnp.min(jnp.where(biased >= mx, lane, E), axis=1, keepdims=True)
    oh = (lane == sel).astype(jnp.float32)
    oh_ref[...] = oh
    score_sel = jnp.sum(oh * scores, axis=1, keepdims=True)
    sc16_ref[...] = score_sel * jnp.ones((1, SW), jnp.float32)

    r = lax.broadcasted_iota(jnp.int32, (BT, BT), 0)
    c = lax.broadcasted_iota(jnp.int32, (BT, BT), 1)
    ltri = (c < r).astype(jnp.float32)

    def chunk(i, carry):
        ohc = oh_ref[pl.ds(i * BT, BT), :]
        rankc = lax.dot_general(ltri, ohc, (((1,), (0,)), ((), ())),
                                preferred_element_type=jnp.float32)
        rankc = rankc + carry
        rank_ref[pl.ds(i * BT, BT), :] = jnp.sum(
            rankc * ohc, axis=1, keepdims=True)
        return carry + jnp.sum(ohc, axis=0, keepdims=True)

    counts_f = lax.fori_loop(0, T // BT, chunk, jnp.zeros((1, E), jnp.float32))
    cnt_ref[...] = counts_f.astype(jnp.int32)

    nblk = jnp.floor((counts_f + (BT - 1)) * (1.0 / BT))
    r8 = lax.broadcasted_iota(jnp.int32, (E, E), 0)
    c8 = lax.broadcasted_iota(jnp.int32, (E, E), 1)
    sut = (r8 < c8).astype(jnp.float32)
    offs = lax.dot_general(nblk, sut, (((1,), (0,)), ((), ())),
                           preferred_element_type=jnp.float32)
    total = jnp.sum(nblk, axis=1, keepdims=True)

    pos_f = jnp.sum(oh_ref[...] * (offs * float(BT)), axis=1,
                    keepdims=True) + rank_ref[...]
    pos_ref[...] = pos_f.astype(jnp.int32)

    jblk = lax.broadcasted_iota(jnp.int32, (NB, E), 0).astype(jnp.float32)
    blk_e_ref[...] = (jnp.sum((jblk >= offs).astype(jnp.float32), axis=1,
                              keepdims=True) - 1.0).astype(jnp.int32)
    jcol = lax.broadcasted_iota(jnp.int32, (NB, 1), 0).astype(jnp.float32)
    blk_a_ref[...] = (jcol < total).astype(jnp.int32)


def _router(xf, gate, bias2d):
    return pl.pallas_call(
        _router_body,
        out_shape=[
            jax.ShapeDtypeStruct((T, 1), jnp.int32),
            jax.ShapeDtypeStruct((T, SW), jnp.float32),
            jax.ShapeDtypeStruct((1, E), jnp.int32),
            jax.ShapeDtypeStruct((NB, 1), jnp.int32),
            jax.ShapeDtypeStruct((NB, 1), jnp.int32),
        ],
        scratch_shapes=[
            pltpu.VMEM((T, E), jnp.float32),
            pltpu.VMEM((T, 1), jnp.float32),
        ],
    )(xf, gate, bias2d)


def _dispatch_body(pos_hbm, xf_hbm, sc_hbm, xs_out, sc_out,
                   idx_v, rows_v, srows_v, sem):
    wid = lax.axis_index("s") * NC + lax.axis_index("c")
    pltpu.sync_copy(pos_hbm.at[wid], idx_v)
    for ci in range(NCH):
        base = wid * TPW + ci * CH
        pltpu.sync_copy(xf_hbm.at[pl.ds(base, CH)], rows_v)
        pltpu.async_copy(rows_v, xs_out.at[idx_v.at[ci]], sem).wait()
        pltpu.sync_copy(sc_hbm.at[pl.ds(base, CH)], srows_v)
        pltpu.async_copy(srows_v, sc_out.at[idx_v.at[ci]], sem).wait()


def _dispatch(pos3, xf, sc16):
    return pl.kernel(
        _dispatch_body,
        out_type=[
            jax.ShapeDtypeStruct((NPAD, D), jnp.float32),
            jax.ShapeDtypeStruct((NPAD, SW), jnp.float32),
        ],
        mesh=plsc.VectorSubcoreMesh(core_axis_name="c", subcore_axis_name="s"),
        scratch_types=[
            pltpu.VMEM((NCH, CH), jnp.int32),
            pltpu.VMEM((CH, D), jnp.float32),
            pltpu.VMEM((CH, SW), jnp.float32),
            pltpu.SemaphoreType.DMA,
        ],
    )(pos3, xf, sc16)


def _ffn_up_body(be_ref, act_ref, xs_ref, sc_ref, w1_ref, w3_ref, h_ref):
    j = pl.program_id(0)

    @pl.when(act_ref[j] != 0)
    def _():
        x = xs_ref[...] * sc_ref[:, 0:1]
        h1 = _dotT(x, w1_ref[0]).astype(jnp.bfloat16)
        g3 = _dotT(x, w3_ref[0]).astype(jnp.bfloat16)
        h_ref[...] = h1 * jax.nn.sigmoid(h1) * g3


def _ffn_down_body(be_ref, act_ref, h_ref, w2_ref, or_ref):
    j = pl.program_id(0)

    @pl.when(act_ref[j] != 0)
    def _():
        o = _dotT(h_ref[...].astype(jnp.float32), w2_ref[0]).astype(jnp.bfloat16)
        or_ref[...] = o.astype(jnp.float32)


def _ffn(blk_e, blk_a, xs_pad, sc_pad, w1, w3, w2):
    up_spec = pltpu.PrefetchScalarGridSpec(
        num_scalar_prefetch=2,
        grid=(NB,),
        in_specs=[
            pl.BlockSpec((BT, D), lambda j, be, act: (j, 0)),
            pl.BlockSpec((BT, SW), lambda j, be, act: (j, 0)),
            pl.BlockSpec((1, H, D), lambda j, be, act: (be[j], 0, 0)),
            pl.BlockSpec((1, H, D), lambda j, be, act: (be[j], 0, 0)),
        ],
        out_specs=pl.BlockSpec((BT, H), lambda j, be, act: (j, 0)),
    )
    h_pad = pl.pallas_call(
        _ffn_up_body,
        grid_spec=up_spec,
        out_shape=jax.ShapeDtypeStruct((NPAD, H), jnp.bfloat16),
    )(blk_e, blk_a, xs_pad, sc_pad, w1, w3)

    down_spec = pltpu.PrefetchScalarGridSpec(
        num_scalar_prefetch=2,
        grid=(NB,),
        in_specs=[
            pl.BlockSpec((BT, H), lambda j, be, act: (j, 0)),
            pl.BlockSpec((1, D, H), lambda j, be, act: (be[j], 0, 0)),
        ],
        out_specs=pl.BlockSpec((BT, D), lambda j, be, act: (j, 0)),
    )
    return pl.pallas_call(
        _ffn_down_body,
        grid_spec=down_spec,
        out_shape=jax.ShapeDtypeStruct((NPAD, D), jnp.float32),
    )(blk_e, blk_a, h_pad, w2)


def _unperm_body(pos_hbm, or_hbm, out_hbm, idx_v, rows_v, sem):
    wid = lax.axis_index("s") * NC + lax.axis_index("c")
    pltpu.sync_copy(pos_hbm.at[wid], idx_v)
    for ci in range(NCH):
        base = wid * TPW + ci * CH
        pltpu.async_copy(or_hbm.at[idx_v.at[ci]], rows_v, sem).wait()
        pltpu.sync_copy(rows_v, out_hbm.at[pl.ds(base, CH)])


def _unpermute(pos3, or_pad):
    return pl.kernel(
        _unperm_body,
        out_type=jax.ShapeDtypeStruct((T, D), jnp.float32),
        mesh=plsc.VectorSubcoreMesh(core_axis_name="c", subcore_axis_name="s"),
        scratch_types=[
            pltpu.VMEM((NCH, CH), jnp.int32),
            pltpu.VMEM((CH, D), jnp.float32),
            pltpu.SemaphoreType.DMA,
        ],
    )(pos3, or_pad)


BTS = 512


def _shared_up_body(xf_ref, w1_ref, w3_ref, hs_ref):
    x = xf_ref[...]
    s1 = _dotT(x, w1_ref[...])
    hs_ref[...] = (s1 * jax.nn.sigmoid(s1) * _dotT(x, w3_ref[...])
                   ).astype(jnp.bfloat16)


def _shared_down_body(hs_ref, w2_ref, or_ref, out_ref):
    out_ref[...] = _dotT(hs_ref[...].astype(jnp.float32),
                         w2_ref[...]) + or_ref[...]


def _shared(xf, sw1, sw3, sw2, or_tok):
    nblk = T // BTS
    hs = pl.pallas_call(
        _shared_up_body,
        grid=(nblk,),
        in_specs=[
            pl.BlockSpec((BTS, D), lambda j: (j, 0)),
            pl.BlockSpec((H, D), lambda j: (0, 0)),
            pl.BlockSpec((H, D), lambda j: (0, 0)),
        ],
        out_specs=pl.BlockSpec((BTS, H), lambda j: (j, 0)),
        out_shape=jax.ShapeDtypeStruct((T, H), jnp.bfloat16),
    )(xf, sw1, sw3)
    return pl.pallas_call(
        _shared_down_body,
        grid=(nblk,),
        in_specs=[
            pl.BlockSpec((BTS, H), lambda j: (j, 0)),
            pl.BlockSpec((D, H), lambda j: (0, 0)),
            pl.BlockSpec((BTS, D), lambda j: (j, 0)),
        ],
        out_specs=pl.BlockSpec((BTS, D), lambda j: (j, 0)),
        out_shape=jax.ShapeDtypeStruct((T, D), jnp.float32),
    )(hs, sw2, or_tok)


def kernel(x, gate, expert_bias, w1, w2, w3, shared_w1, shared_w2, shared_w3):
    bs, slen, dim = x.shape
    xf = x.reshape(T, D)

    pos, sc16, counts, blk_e, blk_a = _router(xf, gate,
                                              expert_bias.reshape(1, E))
    pos3 = pos.reshape(NW, NCH, CH)

    xs_pad, sc_pad = _dispatch(pos3, xf, sc16)

    or_pad = _ffn(blk_e.reshape(NB), blk_a.reshape(NB),
                  xs_pad, sc_pad, w1, w3, w2)

    or_tok = _unpermute(pos3, or_pad)

    out = _shared(xf, shared_w1, shared_w3, shared_w2, or_tok)
    return (out.reshape(bs, slen, dim), counts.reshape(E))


# speedup vs baseline: 1.6960x; 1.0399x over previous
import functools

import jax
import jax.numpy as jnp
from jax import lax
from jax.experimental import pallas as pl
from jax.experimental.pallas import tpu as pltpu
from jax.experimental.pallas import tpu_sc as plsc

T = 2048
D = 2048
H = 1024
E = 8
BT = 512
NB = T // BT + E
NPAD = NB * BT

NC = 2
NS = 16
NW = NC * NS
TPW = T // NW
CH = 16
SW = 128
NCH = TPW // CH


def _dotT(a, b, out_dtype=jnp.float32):
    return lax.dot_general(a, b, (((1,), (1,)), ((), ())),
                           preferred_element_type=out_dtype)


def _router_body(xf_ref, gate_ref, bias_ref, pos_ref, sc16_ref, cnt_ref,
                 blk_e_ref, blk_a_ref, oh_ref, rank_ref):
    xf = xf_ref[...]
    logits = _dotT(xf, gate_ref[...])
    scores = jax.nn.sigmoid(logits)
    biased = scores + bias_ref[...]
    lane = lax.broadcasted_iota(jnp.int32, (T, E), 1)
    mx = jnp.max(biased, axis=1, keepdims=True)
    sel = jnp.min(jnp.where(biased >= mx, lane, E), axis=1, keepdims=True)
    oh = (lane == sel).astype(jnp.float32)
    oh_ref[...] = oh
    score_sel = jnp.sum(oh * scores, axis=1, keepdims=True)
    sc16_ref[...] = score_sel * jnp.ones((1, SW), jnp.float32)

    r = lax.broadcasted_iota(jnp.int32, (BT, BT), 0)
    c = lax.broadcasted_iota(jnp.int32, (BT, BT), 1)
    ltri = (c < r).astype(jnp.float32)

    def chunk(i, carry):
        ohc = oh_ref[pl.ds(i * BT, BT), :]
        rankc = lax.dot_general(ltri, ohc, (((1,), (0,)), ((), ())),
                                preferred_element_type=jnp.float32)
        rankc = rankc + carry
        rank_ref[pl.ds(i * BT, BT), :] = jnp.sum(
            rankc * ohc, axis=1, keepdims=True)
        return carry + jnp.sum(ohc, axis=0, keepdims=True)

    counts_f = lax.fori_loop(0, T // BT, chunk, jnp.zeros((1, E), jnp.float32))
    cnt_ref[...] = counts_f.astype(jnp.int32)

    nblk = jnp.floor((counts_f + (BT - 1)) * (1.0 / BT))
    r8 = lax.broadcasted_iota(jnp.int32, (E, E), 0)
    c8 = lax.broadcasted_iota(jnp.int32, (E, E), 1)
    sut = (r8 < c8).astype(jnp.float32)
    offs = lax.dot_general(nblk, sut, (((1,), (0,)), ((), ())),
                           preferred_element_type=jnp.float32)
    total = jnp.sum(nblk, axis=1, keepdims=True)

    pos_f = jnp.sum(oh_ref[...] * (offs * float(BT)), axis=1,
                    keepdims=True) + rank_ref[...]
    pos_ref[...] = pos_f.astype(jnp.int32)

    jblk = lax.broadcasted_iota(jnp.int32, (NB, E), 0).astype(jnp.float32)
    blk_e_ref[...] = (jnp.sum((jblk >= offs).astype(jnp.float32), axis=1,
                              keepdims=True) - 1.0).astype(jnp.int32)
    jcol = lax.broadcasted_iota(jnp.int32, (NB, 1), 0).astype(jnp.float32)
    blk_a_ref[...] = (jcol < total).astype(jnp.int32)


def _router(xf, gate, bias2d):
    return pl.pallas_call(
        _router_body,
        out_shape=[
            jax.ShapeDtypeStruct((T, 1), jnp.int32),
            jax.ShapeDtypeStruct((T, SW), jnp.float32),
            jax.ShapeDtypeStruct((1, E), jnp.int32),
            jax.ShapeDtypeStruct((NB, 1), jnp.int32),
            jax.ShapeDtypeStruct((NB, 1), jnp.int32),
        ],
        scratch_shapes=[
            pltpu.VMEM((T, E), jnp.float32),
            pltpu.VMEM((T, 1), jnp.float32),
        ],
    )(xf, gate, bias2d)


def _dispatch_body(pos_hbm, xf_hbm, sc_hbm, xs_out, sc_out,
                   idx_v, rows_v, srows_v, sem):
    wid = lax.axis_index("s") * NC + lax.axis_index("c")
    pltpu.sync_copy(pos_hbm.at[wid], idx_v)
    for ci in range(NCH):
        base = wid * TPW + ci * CH
        pltpu.sync_copy(xf_hbm.at[pl.ds(base, CH)], rows_v)
        pltpu.async_copy(rows_v, xs_out.at[idx_v.at[ci]], sem).wait()
        pltpu.sync_copy(sc_hbm.at[pl.ds(base, CH)], srows_v)
        pltpu.async_copy(srows_v, sc_out.at[idx_v.at[ci]], sem).wait()


def _dispatch(pos3, xf, sc16):
    return pl.kernel(
        _dispatch_body,
        out_type=[
            jax.ShapeDtypeStruct((NPAD, D), jnp.float32),
            jax.ShapeDtypeStruct((NPAD, SW), jnp.float32),
        ],
        mesh=plsc.VectorSubcoreMesh(core_axis_name="c", subcore_axis_name="s"),
        scratch_types=[
            pltpu.VMEM((NCH, CH), jnp.int32),
            pltpu.VMEM((CH, D), jnp.float32),
            pltpu.VMEM((CH, SW), jnp.float32),
            pltpu.SemaphoreType.DMA,
        ],
    )(pos3, xf, sc16)


def _ffn_up_body(be_ref, act_ref, xs_ref, sc_ref, w1_ref, w3_ref, hs_dep_ref,
                 h_ref):
    del hs_dep_ref
    j = pl.program_id(0)

    @pl.when(act_ref[j] != 0)
    def _():
        x = xs_ref[...] * sc_ref[:, 0:1]
        h1 = _dotT(x, w1_ref[0]).astype(jnp.bfloat16)
        g3 = _dotT(x, w3_ref[0]).astype(jnp.bfloat16)
        h_ref[...] = h1 * jax.nn.sigmoid(h1) * g3


def _ffn_down_body(be_ref, act_ref, h_ref, w2_ref, or_ref):
    j = pl.program_id(0)

    @pl.when(act_ref[j] != 0)
    def _():
        o = _dotT(h_ref[...].astype(jnp.float32), w2_ref[0]).astype(jnp.bfloat16)
        or_ref[...] = o.astype(jnp.float32)


def _ffn(blk_e, blk_a, xs_pad, sc_pad, w1, w3, w2, hs_dep):
    up_spec = pltpu.PrefetchScalarGridSpec(
        num_scalar_prefetch=2,
        grid=(NB,),
        in_specs=[
            pl.BlockSpec((BT, D), lambda j, be, act: (j, 0)),
            pl.BlockSpec((BT, SW), lambda j, be, act: (j, 0)),
            pl.BlockSpec((1, H, D), lambda j, be, act: (be[j], 0, 0)),
            pl.BlockSpec((1, H, D), lambda j, be, act: (be[j], 0, 0)),
            pl.BlockSpec((8, 128), lambda j, be, act: (0, 0)),
        ],
        out_specs=pl.BlockSpec((BT, H), lambda j, be, act: (j, 0)),
    )
    h_pad = pl.pallas_call(
        _ffn_up_body,
        grid_spec=up_spec,
        out_shape=jax.ShapeDtypeStruct((NPAD, H), jnp.bfloat16),
    )(blk_e, blk_a, xs_pad, sc_pad, w1, w3, hs_dep)

    down_spec = pltpu.PrefetchScalarGridSpec(
        num_scalar_prefetch=2,
        grid=(NB,),
        in_specs=[
            pl.BlockSpec((BT, H), lambda j, be, act: (j, 0)),
            pl.BlockSpec((1, D, H), lambda j, be, act: (be[j], 0, 0)),
        ],
        out_specs=pl.BlockSpec((BT, D), lambda j, be, act: (j, 0)),
    )
    return pl.pallas_call(
        _ffn_down_body,
        grid_spec=down_spec,
        out_shape=jax.ShapeDtypeStruct((NPAD, D), jnp.float32),
    )(blk_e, blk_a, h_pad, w2)


def _unperm_body(pos_hbm, or_hbm, out_hbm, idx_v, rows_v, sem):
    wid = lax.axis_index("s") * NC + lax.axis_index("c")
    pltpu.sync_copy(pos_hbm.at[wid], idx_v)
    for ci in range(NCH):
        base = wid * TPW + ci * CH
        pltpu.async_copy(or_hbm.at[idx_v.at[ci]], rows_v, sem).wait()
        pltpu.sync_copy(rows_v, out_hbm.at[pl.ds(base, CH)])


def _unpermute(pos3, or_pad):
    return pl.kernel(
        _unperm_body,
        out_type=jax.ShapeDtypeStruct((T, D), jnp.float32),
        mesh=plsc.VectorSubcoreMesh(core_axis_name="c", subcore_axis_name="s"),
        scratch_types=[
            pltpu.VMEM((NCH, CH), jnp.int32),
            pltpu.VMEM((CH, D), jnp.float32),
            pltpu.SemaphoreType.DMA,
        ],
    )(pos3, or_pad)


BTS = 512


def _shared_up_body(xf_ref, w1_ref, w3_ref, hs_ref):
    x = xf_ref[...]
    s1 = _dotT(x, w1_ref[...])
    hs_ref[...] = (s1 * jax.nn.sigmoid(s1) * _dotT(x, w3_ref[...])
                   ).astype(jnp.bfloat16)


def _shared_down_body(hs_ref, w2_ref, or_ref, out_ref):
    out_ref[...] = _dotT(hs_ref[...].astype(jnp.float32),
                         w2_ref[...]) + or_ref[...]


def _shared_up(xf, sw1, sw3):
    nblk = T // BTS
    return pl.pallas_call(
        _shared_up_body,
        grid=(nblk,),
        in_specs=[
            pl.BlockSpec((BTS, D), lambda j: (j, 0)),
            pl.BlockSpec((H, D), lambda j: (0, 0)),
            pl.BlockSpec((H, D), lambda j: (0, 0)),
        ],
        out_specs=pl.BlockSpec((BTS, H), lambda j: (j, 0)),
        out_shape=jax.ShapeDtypeStruct((T, H), jnp.bfloat16),
    )(xf, sw1, sw3)


def _shared_down(hs, sw2, or_tok):
    nblk = T // BTS
    return pl.pallas_call(
        _shared_down_body,
        grid=(nblk,),
        in_specs=[
            pl.BlockSpec((BTS, H), lambda j: (j, 0)),
            pl.BlockSpec((D, H), lambda j: (0, 0)),
            pl.BlockSpec((BTS, D), lambda j: (j, 0)),
        ],
        out_specs=pl.BlockSpec((BTS, D), lambda j: (j, 0)),
        out_shape=jax.ShapeDtypeStruct((T, D), jnp.float32),
    )(hs, sw2, or_tok)


def kernel(x, gate, expert_bias, w1, w2, w3, shared_w1, shared_w2, shared_w3):
    bs, slen, dim = x.shape
    xf = x.reshape(T, D)

    pos, sc16, counts, blk_e, blk_a = _router(xf, gate,
                                              expert_bias.reshape(1, E))
    pos3 = pos.reshape(NW, NCH, CH)

    xs_pad, sc_pad = _dispatch(pos3, xf, sc16)

    hs = _shared_up(xf, shared_w1, shared_w3)
    or_pad = _ffn(blk_e.reshape(NB), blk_a.reshape(NB),
                  xs_pad, sc_pad, w1, w3, w2, hs)

    or_tok = _unpermute(pos3, or_pad)

    out = _shared_down(hs, shared_w2, or_tok)
    return (out.reshape(bs, slen, dim), counts.reshape(E))


# speedup vs baseline: 1.7625x; 1.0392x over previous
import functools

import jax
import jax.numpy as jnp
from jax import lax
from jax.experimental import pallas as pl
from jax.experimental.pallas import tpu as pltpu
from jax.experimental.pallas import tpu_sc as plsc

T = 2048
D = 2048
H = 1024
E = 8
BT = 384
NB = -(-T // BT) + E
NPAD = NB * BT
RCH = 512

NC = 2
NS = 16
NW = NC * NS
TPW = T // NW
CH = 16
SW = 128
NCH = TPW // CH


def _dotT(a, b, out_dtype=jnp.float32):
    return lax.dot_general(a, b, (((1,), (1,)), ((), ())),
                           preferred_element_type=out_dtype)


def _router_body(xf_ref, gate_ref, bias_ref, pos_ref, sc16_ref, cnt_ref,
                 blk_e_ref, blk_a_ref, oh_ref, rank_ref):
    xf = xf_ref[...]
    logits = _dotT(xf, gate_ref[...])
    scores = jax.nn.sigmoid(logits)
    biased = scores + bias_ref[...]
    lane = lax.broadcasted_iota(jnp.int32, (T, E), 1)
    mx = jnp.max(biased, axis=1, keepdims=True)
    sel = jnp.min(jnp.where(biased >= mx, lane, E), axis=1, keepdims=True)
    oh = (lane == sel).astype(jnp.float32)
    oh_ref[...] = oh
    score_sel = jnp.sum(oh * scores, axis=1, keepdims=True)
    sc16_ref[...] = score_sel * jnp.ones((1, SW), jnp.float32)

    r = lax.broadcasted_iota(jnp.int32, (RCH, RCH), 0)
    c = lax.broadcasted_iota(jnp.int32, (RCH, RCH), 1)
    ltri = (c < r).astype(jnp.float32)

    def chunk(i, carry):
        ohc = oh_ref[pl.ds(i * RCH, RCH), :]
        rankc = lax.dot_general(ltri, ohc, (((1,), (0,)), ((), ())),
                                preferred_element_type=jnp.float32)
        rankc = rankc + carry
        rank_ref[pl.ds(i * RCH, RCH), :] = jnp.sum(
            rankc * ohc, axis=1, keepdims=True)
        return carry + jnp.sum(ohc, axis=0, keepdims=True)

    counts_f = lax.fori_loop(0, T // RCH, chunk, jnp.zeros((1, E), jnp.float32))
    cnt_ref[...] = counts_f.astype(jnp.int32)

    nblk = jnp.floor((counts_f + (BT - 1)) / float(BT))
    r8 = lax.broadcasted_iota(jnp.int32, (E, E), 0)
    c8 = lax.broadcasted_iota(jnp.int32, (E, E), 1)
    sut = (r8 < c8).astype(jnp.float32)
    offs = lax.dot_general(nblk, sut, (((1,), (0,)), ((), ())),
                           preferred_element_type=jnp.float32)
    total = jnp.sum(nblk, axis=1, keepdims=True)

    pos_f = jnp.sum(oh_ref[...] * (offs * float(BT)), axis=1,
                    keepdims=True) + rank_ref[...]
    pos_ref[...] = pos_f.astype(jnp.int32)

    jblk = lax.broadcasted_iota(jnp.int32, (NB, E), 0).astype(jnp.float32)
    blk_e_ref[...] = (jnp.sum((jblk >= offs).astype(jnp.float32), axis=1,
                              keepdims=True) - 1.0).astype(jnp.int32)
    jcol = lax.broadcasted_iota(jnp.int32, (NB, 1), 0).astype(jnp.float32)
    blk_a_ref[...] = (jcol < total).astype(jnp.int32)


def _router(xf, gate, bias2d):
    return pl.pallas_call(
        _router_body,
        out_shape=[
            jax.ShapeDtypeStruct((T, 1), jnp.int32),
            jax.ShapeDtypeStruct((T, SW), jnp.float32),
            jax.ShapeDtypeStruct((1, E), jnp.int32),
            jax.ShapeDtypeStruct((NB, 1), jnp.int32),
            jax.ShapeDtypeStruct((NB, 1), jnp.int32),
        ],
        scratch_shapes=[
            pltpu.VMEM((T, E), jnp.float32),
            pltpu.VMEM((T, 1), jnp.float32),
        ],
    )(xf, gate, bias2d)


def _dispatch_body(pos_hbm, xf_hbm, sc_hbm, xs_out, sc_out,
                   idx_v, rows_v, srows_v, sem):
    wid = lax.axis_index("s") * NC + lax.axis_index("c")
    pltpu.sync_copy(pos_hbm.at[wid], idx_v)
    for ci in range(NCH):
        base = wid * TPW + ci * CH
        pltpu.sync_copy(xf_hbm.at[pl.ds(base, CH)], rows_v)
        pltpu.async_copy(rows_v, xs_out.at[idx_v.at[ci]], sem).wait()
        pltpu.sync_copy(sc_hbm.at[pl.ds(base, CH)], srows_v)
        pltpu.async_copy(srows_v, sc_out.at[idx_v.at[ci]], sem).wait()


def _dispatch(pos3, xf, sc16):
    return pl.kernel(
        _dispatch_body,
        out_type=[
            jax.ShapeDtypeStruct((NPAD, D), jnp.float32),
            jax.ShapeDtypeStruct((NPAD, SW), jnp.float32),
        ],
        mesh=plsc.VectorSubcoreMesh(core_axis_name="c", subcore_axis_name="s"),
        scratch_types=[
            pltpu.VMEM((NCH, CH), jnp.int32),
            pltpu.VMEM((CH, D), jnp.float32),
            pltpu.VMEM((CH, SW), jnp.float32),
            pltpu.SemaphoreType.DMA,
        ],
    )(pos3, xf, sc16)


def _ffn_up_body(be_ref, act_ref, xs_ref, sc_ref, w1_ref, w3_ref, hs_dep_ref,
                 h_ref):
    del hs_dep_ref
    j = pl.program_id(0)

    @pl.when(act_ref[j] != 0)
    def _():
        x = xs_ref[...] * sc_ref[:, 0:1]
        h1 = _dotT(x, w1_ref[0]).astype(jnp.bfloat16)
        g3 = _dotT(x, w3_ref[0]).astype(jnp.bfloat16)
        h_ref[...] = h1 * jax.nn.sigmoid(h1) * g3


def _ffn_down_body(be_ref, act_ref, h_ref, w2_ref, or_ref):
    j = pl.program_id(0)

    @pl.when(act_ref[j] != 0)
    def _():
        o = _dotT(h_ref[...].astype(jnp.float32), w2_ref[0]).astype(jnp.bfloat16)
        or_ref[...] = o.astype(jnp.float32)


def _ffn(blk_e, blk_a, xs_pad, sc_pad, w1, w3, w2, hs_dep):
    up_spec = pltpu.PrefetchScalarGridSpec(
        num_scalar_prefetch=2,
        grid=(NB,),
        in_specs=[
            pl.BlockSpec((BT, D), lambda j, be, act: (j, 0)),
            pl.BlockSpec((BT, SW), lambda j, be, act: (j, 0)),
            pl.BlockSpec((1, H, D), lambda j, be, act: (be[j], 0, 0)),
            pl.BlockSpec((1, H, D), lambda j, be, act: (be[j], 0, 0)),
            pl.BlockSpec((8, 128), lambda j, be, act: (0, 0)),
        ],
        out_specs=pl.BlockSpec((BT, H), lambda j, be, act: (j, 0)),
    )
    h_pad = pl.pallas_call(
        _ffn_up_body,
        grid_spec=up_spec,
        out_shape=jax.ShapeDtypeStruct((NPAD, H), jnp.bfloat16),
    )(blk_e, blk_a, xs_pad, sc_pad, w1, w3, hs_dep)

    down_spec = pltpu.PrefetchScalarGridSpec(
        num_scalar_prefetch=2,
        grid=(NB,),
        in_specs=[
            pl.BlockSpec((BT, H), lambda j, be, act: (j, 0)),
            pl.BlockSpec((1, D, H), lambda j, be, act: (be[j], 0, 0)),
        ],
        out_specs=pl.BlockSpec((BT, D), lambda j, be, act: (j, 0)),
    )
    return pl.pallas_call(
        _ffn_down_body,
        grid_spec=down_spec,
        out_shape=jax.ShapeDtypeStruct((NPAD, D), jnp.float32),
    )(blk_e, blk_a, h_pad, w2)


def _unperm_body(pos_hbm, or_hbm, out_hbm, idx_v, rows_v, sem):
    wid = lax.axis_index("s") * NC + lax.axis_index("c")
    pltpu.sync_copy(pos_hbm.at[wid], idx_v)
    for ci in range(NCH):
        base = wid * TPW + ci * CH
        pltpu.async_copy(or_hbm.at[idx_v.at[ci]], rows_v, sem).wait()
        pltpu.sync_copy(rows_v, out_hbm.at[pl.ds(base, CH)])


def _unpermute(pos3, or_pad):
    return pl.kernel(
        _unperm_body,
        out_type=jax.ShapeDtypeStruct((T, D), jnp.float32),
        mesh=plsc.VectorSubcoreMesh(core_axis_name="c", subcore_axis_name="s"),
        scratch_types=[
            pltpu.VMEM((NCH, CH), jnp.int32),
            pltpu.VMEM((CH, D), jnp.float32),
            pltpu.SemaphoreType.DMA,
        ],
    )(pos3, or_pad)


BTS = 512


def _shared_up_body(xf_ref, w1_ref, w3_ref, hs_ref):
    x = xf_ref[...]
    s1 = _dotT(x, w1_ref[...])
    hs_ref[...] = (s1 * jax.nn.sigmoid(s1) * _dotT(x, w3_ref[...])
                   ).astype(jnp.bfloat16)


def _shared_down_body(hs_ref, w2_ref, or_ref, out_ref):
    out_ref[...] = _dotT(hs_ref[...].astype(jnp.float32),
                         w2_ref[...]) + or_ref[...]


def _shared_up(xf, sw1, sw3):
    nblk = T // BTS
    return pl.pallas_call(
        _shared_up_body,
        grid=(nblk,),
        in_specs=[
            pl.BlockSpec((BTS, D), lambda j: (j, 0)),
            pl.BlockSpec((H, D), lambda j: (0, 0)),
            pl.BlockSpec((H, D), lambda j: (0, 0)),
        ],
        out_specs=pl.BlockSpec((BTS, H), lambda j: (j, 0)),
        out_shape=jax.ShapeDtypeStruct((T, H), jnp.bfloat16),
    )(xf, sw1, sw3)


def _shared_down(hs, sw2, or_tok):
    nblk = T // BTS
    return pl.pallas_call(
        _shared_down_body,
        grid=(nblk,),
        in_specs=[
            pl.BlockSpec((BTS, H), lambda j: (j, 0)),
            pl.BlockSpec((D, H), lambda j: (0, 0)),
            pl.BlockSpec((BTS, D), lambda j: (j, 0)),
        ],
        out_specs=pl.BlockSpec((BTS, D), lambda j: (j, 0)),
        out_shape=jax.ShapeDtypeStruct((T, D), jnp.float32),
    )(hs, sw2, or_tok)


def kernel(x, gate, expert_bias, w1, w2, w3, shared_w1, shared_w2, shared_w3):
    bs, slen, dim = x.shape
    xf = x.reshape(T, D)

    pos, sc16, counts, blk_e, blk_a = _router(xf, gate,
                                              expert_bias.reshape(1, E))
    pos3 = pos.reshape(NW, NCH, CH)

    xs_pad, sc_pad = _dispatch(pos3, xf, sc16)

    hs = _shared_up(xf, shared_w1, shared_w3)
    or_pad = _ffn(blk_e.reshape(NB), blk_a.reshape(NB),
                  xs_pad, sc_pad, w1, w3, w2, hs)

    or_tok = _unpermute(pos3, or_pad)

    out = _shared_down(hs, shared_w2, or_tok)
    return (out.reshape(bs, slen, dim), counts.reshape(E))


# speedup vs baseline: 1.7724x; 1.0056x over previous
import jax
import jax.numpy as jnp
from jax import lax
from jax.experimental import pallas as pl
from jax.experimental.pallas import tpu as pltpu
from jax.experimental.pallas import tpu_sc as plsc

T = 2048
D = 2048
H = 1024
E = 8
BT = 384
NB = -(-T // BT) + E
NPAD = NB * BT
RCH = 512

NC = 2
NS = 16
NW = NC * NS
TPW = T // NW
CH = 32
SW = 128
NCH = TPW // CH


def _dotT(a, b, out_dtype=jnp.float32):
    return lax.dot_general(a, b, (((1,), (1,)), ((), ())),
                           preferred_element_type=out_dtype)


def _router_body(xf_ref, gate_ref, bias_ref, pos_ref, sc16_ref, cnt_ref,
                 blk_e_ref, blk_a_ref, oh_ref, rank_ref):
    xf = xf_ref[...]
    logits = _dotT(xf, gate_ref[...])
    scores = jax.nn.sigmoid(logits)
    biased = scores + bias_ref[...]
    lane = lax.broadcasted_iota(jnp.int32, (T, E), 1)
    mx = jnp.max(biased, axis=1, keepdims=True)
    sel = jnp.min(jnp.where(biased >= mx, lane, E), axis=1, keepdims=True)
    oh = (lane == sel).astype(jnp.float32)
    oh_ref[...] = oh
    score_sel = jnp.sum(oh * scores, axis=1, keepdims=True)
    sc16_ref[...] = score_sel * jnp.ones((1, SW), jnp.float32)

    r = lax.broadcasted_iota(jnp.int32, (RCH, RCH), 0)
    c = lax.broadcasted_iota(jnp.int32, (RCH, RCH), 1)
    ltri = (c < r).astype(jnp.float32)

    def chunk(i, carry):
        ohc = oh_ref[pl.ds(i * RCH, RCH), :]
        rankc = lax.dot_general(ltri, ohc, (((1,), (0,)), ((), ())),
                                preferred_element_type=jnp.float32)
        rankc = rankc + carry
        rank_ref[pl.ds(i * RCH, RCH), :] = jnp.sum(
            rankc * ohc, axis=1, keepdims=True)
        return carry + jnp.sum(ohc, axis=0, keepdims=True)

    counts_f = lax.fori_loop(0, T // RCH, chunk, jnp.zeros((1, E), jnp.float32))
    cnt_ref[...] = counts_f.astype(jnp.int32)

    nblk = jnp.floor((counts_f + (BT - 1)) / float(BT))
    r8 = lax.broadcasted_iota(jnp.int32, (E, E), 0)
    c8 = lax.broadcasted_iota(jnp.int32, (E, E), 1)
    sut = (r8 < c8).astype(jnp.float32)
    offs = lax.dot_general(nblk, sut, (((1,), (0,)), ((), ())),
                           preferred_element_type=jnp.float32)
    total = jnp.sum(nblk, axis=1, keepdims=True)

    pos_f = jnp.sum(oh_ref[...] * (offs * float(BT)), axis=1,
                    keepdims=True) + rank_ref[...]
    pos_ref[...] = pos_f.astype(jnp.int32)

    jblk = lax.broadcasted_iota(jnp.int32, (NB, E), 0).astype(jnp.float32)
    blk_e_ref[...] = (jnp.sum((jblk >= offs).astype(jnp.float32), axis=1,
                              keepdims=True) - 1.0).astype(jnp.int32)
    jcol = lax.broadcasted_iota(jnp.int32, (NB, 1), 0).astype(jnp.float32)
    blk_a_ref[...] = (jcol < total).astype(jnp.int32)


def _router(xf, gate, bias2d):
    return pl.pallas_call(
        _router_body,
        out_shape=[
            jax.ShapeDtypeStruct((T, 1), jnp.int32),
            jax.ShapeDtypeStruct((T, SW), jnp.float32),
            jax.ShapeDtypeStruct((1, E), jnp.int32),
            jax.ShapeDtypeStruct((NB, 1), jnp.int32),
            jax.ShapeDtypeStruct((NB, 1), jnp.int32),
        ],
        scratch_shapes=[
            pltpu.VMEM((T, E), jnp.float32),
            pltpu.VMEM((T, 1), jnp.float32),
        ],
    )(xf, gate, bias2d)


def _dispatch_body(pos_hbm, xf_hbm, sc_hbm, xs_out, sc_out,
                   idx_v, rows_v, srows_v, sem):
    wid = lax.axis_index("s") * NC + lax.axis_index("c")
    pltpu.sync_copy(pos_hbm.at[wid], idx_v)
    for ci in range(NCH):
        base = wid * TPW + ci * CH
        pltpu.sync_copy(xf_hbm.at[pl.ds(base, CH)], rows_v)
        pltpu.async_copy(rows_v, xs_out.at[idx_v.at[ci]], sem).wait()
        pltpu.sync_copy(sc_hbm.at[pl.ds(base, CH)], srows_v)
        pltpu.async_copy(srows_v, sc_out.at[idx_v.at[ci]], sem).wait()


def _dispatch(pos3, xf, sc16):
    return pl.kernel(
        _dispatch_body,
        out_type=[
            jax.ShapeDtypeStruct((NPAD, D), jnp.float32),
            jax.ShapeDtypeStruct((NPAD, SW), jnp.float32),
        ],
        mesh=plsc.VectorSubcoreMesh(core_axis_name="c", subcore_axis_name="s"),
        scratch_types=[
            pltpu.VMEM((NCH, CH), jnp.int32),
            pltpu.VMEM((CH, D), jnp.float32),
            pltpu.VMEM((CH, SW), jnp.float32),
            pltpu.SemaphoreType.DMA,
        ],
    )(pos3, xf, sc16)


def _ffn_up_body(be_ref, act_ref, xs_ref, sc_ref, w1_ref, w3_ref, hs_dep_ref,
                 h_ref):
    del hs_dep_ref
    j = pl.program_id(0)

    @pl.when(act_ref[j] != 0)
    def _():
        x = xs_ref[...] * sc_ref[:, 0:1]
        h1 = _dotT(x, w1_ref[0]).astype(jnp.bfloat16)
        g3 = _dotT(x, w3_ref[0]).astype(jnp.bfloat16)
        h_ref[...] = h1 * jax.nn.sigmoid(h1) * g3


def _ffn_down_body(be_ref, act_ref, h_ref, w2_ref, or_ref):
    j = pl.program_id(0)

    @pl.when(act_ref[j] != 0)
    def _():
        o = _dotT(h_ref[...].astype(jnp.float32), w2_ref[0]).astype(jnp.bfloat16)
        or_ref[...] = o.astype(jnp.float32)


def _ffn(blk_e, blk_a, xs_pad, sc_pad, w1, w3, w2, hs_dep):
    up_spec = pltpu.PrefetchScalarGridSpec(
        num_scalar_prefetch=2,
        grid=(NB,),
        in_specs=[
            pl.BlockSpec((BT, D), lambda j, be, act: (j, 0)),
            pl.BlockSpec((BT, SW), lambda j, be, act: (j, 0)),
            pl.BlockSpec((1, H, D), lambda j, be, act: (be[j], 0, 0)),
            pl.BlockSpec((1, H, D), lambda j, be, act: (be[j], 0, 0)),
            pl.BlockSpec((8, 128), lambda j, be, act: (0, 0)),
        ],
        out_specs=pl.BlockSpec((BT, H), lambda j, be, act: (j, 0)),
    )
    h_pad = pl.pallas_call(
        _ffn_up_body,
        grid_spec=up_spec,
        out_shape=jax.ShapeDtypeStruct((NPAD, H), jnp.bfloat16),
    )(blk_e, blk_a, xs_pad, sc_pad, w1, w3, hs_dep)

    down_spec = pltpu.PrefetchScalarGridSpec(
        num_scalar_prefetch=2,
        grid=(NB,),
        in_specs=[
            pl.BlockSpec((BT, H), lambda j, be, act: (j, 0)),
            pl.BlockSpec((1, D, H), lambda j, be, act: (be[j], 0, 0)),
        ],
        out_specs=pl.BlockSpec((BT, D), lambda j, be, act: (j, 0)),
    )
    return pl.pallas_call(
        _ffn_down_body,
        grid_spec=down_spec,
        out_shape=jax.ShapeDtypeStruct((NPAD, D), jnp.float32),
    )(blk_e, blk_a, h_pad, w2)


def _unperm_body(pos_hbm, or_hbm, out_hbm, idx_v, rows_v, sem):
    wid = lax.axis_index("s") * NC + lax.axis_index("c")
    pltpu.sync_copy(pos_hbm.at[wid], idx_v)
    for ci in range(NCH):
        base = wid * TPW + ci * CH
        pltpu.async_copy(or_hbm.at[idx_v.at[ci]], rows_v, sem).wait()
        pltpu.sync_copy(rows_v, out_hbm.at[pl.ds(base, CH)])


def _unpermute(pos3, or_pad):
    return pl.kernel(
        _unperm_body,
        out_type=jax.ShapeDtypeStruct((T, D), jnp.float32),
        mesh=plsc.VectorSubcoreMesh(core_axis_name="c", subcore_axis_name="s"),
        scratch_types=[
            pltpu.VMEM((NCH, CH), jnp.int32),
            pltpu.VMEM((CH, D), jnp.float32),
            pltpu.SemaphoreType.DMA,
        ],
    )(pos3, or_pad)


BTS = 512


def _shared_up_body(xf_ref, w1_ref, w3_ref, hs_ref):
    x = xf_ref[...]
    s1 = _dotT(x, w1_ref[...])
    hs_ref[...] = (s1 * jax.nn.sigmoid(s1) * _dotT(x, w3_ref[...])
                   ).astype(jnp.bfloat16)


def _shared_down_body(hs_ref, w2_ref, or_ref, out_ref):
    out_ref[...] = _dotT(hs_ref[...].astype(jnp.float32),
                         w2_ref[...]) + or_ref[...]


def _shared_up(xf, sw1, sw3):
    nblk = T // BTS
    return pl.pallas_call(
        _shared_up_body,
        grid=(nblk,),
        in_specs=[
            pl.BlockSpec((BTS, D), lambda j: (j, 0)),
            pl.BlockSpec((H, D), lambda j: (0, 0)),
            pl.BlockSpec((H, D), lambda j: (0, 0)),
        ],
        out_specs=pl.BlockSpec((BTS, H), lambda j: (j, 0)),
        out_shape=jax.ShapeDtypeStruct((T, H), jnp.bfloat16),
    )(xf, sw1, sw3)


def _shared_down(hs, sw2, or_tok):
    nblk = T // BTS
    return pl.pallas_call(
        _shared_down_body,
        grid=(nblk,),
        in_specs=[
            pl.BlockSpec((BTS, H), lambda j: (j, 0)),
            pl.BlockSpec((D, H), lambda j: (0, 0)),
            pl.BlockSpec((BTS, D), lambda j: (j, 0)),
        ],
        out_specs=pl.BlockSpec((BTS, D), lambda j: (j, 0)),
        out_shape=jax.ShapeDtypeStruct((T, D), jnp.float32),
    )(hs, sw2, or_tok)


def kernel(x, gate, expert_bias, w1, w2, w3, shared_w1, shared_w2, shared_w3):
    bs, slen, dim = x.shape
    xf = x.reshape(T, D)

    pos, sc16, counts, blk_e, blk_a = _router(xf, gate,
                                              expert_bias.reshape(1, E))
    pos3 = pos.reshape(NW, NCH, CH)

    xs_pad, sc_pad = _dispatch(pos3, xf, sc16)

    hs = _shared_up(xf, shared_w1, shared_w3)
    or_pad = _ffn(blk_e.reshape(NB), blk_a.reshape(NB),
                  xs_pad, sc_pad, w1, w3, w2, hs)

    or_tok = _unpermute(pos3, or_pad)

    out = _shared_down(hs, shared_w2, or_tok)
    return (out.reshape(bs, slen, dim), counts.reshape(E))
